# Initial kernel scaffold; baseline (speedup 1.0000x reference)
import functools

import jax
import jax.numpy as jnp
from jax import lax
from jax.experimental import pallas as pl
from jax.experimental.pallas import tpu as pltpu
from jax.experimental.pallas import tpu_sc as plsc

N = 10000
E = 320000
D = 128

NC = 2
NS = 16
NW = NC * NS
K = 128
CW = 80
EPW = K * CW
E_PAD = EPW * NW

A_ROWS = 10048
ZROWS_PER_TILE = A_ROWS // NS
ZB = 157
OUT_PER_TILE = N // NS
DUMMY_DST = N


def _segsum_body(table, srcs, dsts, out, src_v, dst_v, rows_v, zbuf, sem, accum):
    c = lax.axis_index("c")
    s = lax.axis_index("s")
    wid = s * NC + c

    zeros16 = jnp.zeros((16,), jnp.float32)

    def _zrow(i, _):
        def _zcol(j, __):
            zbuf[i, pl.ds(j * 16, 16)] = zeros16
            return 0
        lax.fori_loop(0, D // 16, _zcol, 0)
        return 0

    lax.fori_loop(0, ZB, _zrow, 0)
    for t in range(ZROWS_PER_TILE // ZB):
        pltpu.sync_copy(zbuf, accum.at[pl.ds(s * ZROWS_PER_TILE + t * ZB, ZB)])

    plsc.subcore_barrier()

    pltpu.sync_copy(srcs.at[wid], src_v)
    pltpu.sync_copy(dsts.at[wid], dst_v)

    def _chunk(j, _):
        pltpu.async_copy(table.at[src_v.at[j]], rows_v, sem).wait()
        pltpu.sync_copy(rows_v, accum.at[dst_v.at[j]], add=True)
        return 0

    lax.fori_loop(0, CW, _chunk, 0)

    plsc.subcore_barrier()

    pltpu.sync_copy(
        accum.at[pl.ds(s * OUT_PER_TILE, OUT_PER_TILE)],
        out.at[c, pl.ds(s * OUT_PER_TILE, OUT_PER_TILE)],
    )


_segsum = pl.kernel(
    _segsum_body,
    out_type=jax.ShapeDtypeStruct((NC, N, D), jnp.float32),
    mesh=plsc.VectorSubcoreMesh(core_axis_name="c", subcore_axis_name="s"),
    scratch_types=[
        pltpu.VMEM((CW, K), jnp.int32),
        pltpu.VMEM((CW, K), jnp.int32),
        pltpu.VMEM((K, D), jnp.float32),
        pltpu.VMEM((ZB, D), jnp.float32),
        pltpu.SemaphoreType.DMA,
        pltpu.VMEM_SHARED((A_ROWS, D), jnp.float32),
    ],
)


def _linear_block(a_ref, b_ref, w_ref, bias_ref, o_ref, *, relu):
    x = a_ref[...] + b_ref[...]
    y = lax.dot_general(x, w_ref[...], (((1,), (1,)), ((), ())),
                        preferred_element_type=jnp.float32)
    y = y + bias_ref[...]
    if relu:
        y = jnp.maximum(y, 0.0)
    o_ref[...] = y


def _linear(parts, w, bias, relu):
    blk = 2000
    return pl.pallas_call(
        functools.partial(_linear_block, relu=relu),
        grid=(N // blk,),
        in_specs=[
            pl.BlockSpec((blk, D), lambda i: (i, 0)),
            pl.BlockSpec((blk, D), lambda i: (i, 0)),
            pl.BlockSpec((D, D), lambda i: (0, 0)),
            pl.BlockSpec((1, D), lambda i: (0, 0)),
        ],
        out_specs=pl.BlockSpec((blk, D), lambda i: (i, 0)),
        out_shape=jax.ShapeDtypeStruct((N, D), jnp.float32),
    )(parts[0], parts[1], w, bias.reshape(1, D))


def kernel(h, edge_index, W1, b1, W2, b2):
    src = edge_index[0].astype(jnp.int32)
    dst = edge_index[1].astype(jnp.int32)
    pad = E_PAD - E
    srcs = jnp.concatenate([src, jnp.zeros((pad,), jnp.int32)]).reshape(NW, CW, K)
    dsts = jnp.concatenate([dst, jnp.full((pad,), DUMMY_DST, jnp.int32)]).reshape(NW, CW, K)

    parts0 = _segsum(h, srcs, dsts)
    h1 = _linear(parts0, W1, b1, relu=True)
    parts1 = _segsum(h1, srcs, dsts)
    return _linear(parts1, W2, b2, relu=False)


# speedup vs baseline: 3.1334x; 3.1334x over previous
import functools

import jax
import jax.numpy as jnp
from jax import lax
from jax.experimental import pallas as pl
from jax.experimental.pallas import tpu as pltpu
from jax.experimental.pallas import tpu_sc as plsc

N = 10000
E = 320000
D = 128

NC = 2
NS = 16
NW = NC * NS
K = 128
CW = 80
EPW = K * CW
E_PAD = EPW * NW

A_ROWS = 10240
ZROWS_PER_TILE = A_ROWS // NS
ZB = 64
OUT_PER_TILE = 624
DUMMY_DST = N


def _segsum_body(table, srcs, dsts, out, src_v, dst_v, rows_v, zbuf, sem, accum):
    c = lax.axis_index("c")
    s = lax.axis_index("s")
    wid = s * NC + c

    zeros16 = jnp.zeros((16,), jnp.float32)

    def _zrow(i, _):
        def _zcol(j, __):
            zbuf[i, pl.ds(j * 16, 16)] = zeros16
            return 0
        lax.fori_loop(0, D // 16, _zcol, 0)
        return 0

    lax.fori_loop(0, ZB, _zrow, 0)

    def _zcopy(t, _):
        pltpu.sync_copy(zbuf, accum.at[pl.ds(s * ZROWS_PER_TILE + t * ZB, ZB)])
        return 0

    lax.fori_loop(0, ZROWS_PER_TILE // ZB, _zcopy, 0)

    plsc.subcore_barrier()

    pltpu.sync_copy(srcs.at[wid], src_v)
    pltpu.sync_copy(dsts.at[wid], dst_v)

    def _chunk(j, _):
        pltpu.async_copy(table.at[src_v.at[j]], rows_v, sem).wait()
        pltpu.sync_copy(rows_v, accum.at[dst_v.at[j]], add=True)
        return 0

    lax.fori_loop(0, CW, _chunk, 0)

    plsc.subcore_barrier()

    @pl.when(s < NS - 1)
    def _():
        pltpu.sync_copy(
            accum.at[pl.ds(s * OUT_PER_TILE, OUT_PER_TILE)],
            out.at[c, pl.ds(s * OUT_PER_TILE, OUT_PER_TILE)],
        )

    @pl.when(s == NS - 1)
    def _():
        tail = N - (NS - 1) * OUT_PER_TILE
        base = (NS - 1) * OUT_PER_TILE
        pltpu.sync_copy(
            accum.at[pl.ds(base, tail)],
            out.at[c, pl.ds(base, tail)],
        )


_segsum = pl.kernel(
    _segsum_body,
    out_type=jax.ShapeDtypeStruct((NC, N, D), jnp.float32),
    mesh=plsc.VectorSubcoreMesh(core_axis_name="c", subcore_axis_name="s"),
    scratch_types=[
        pltpu.VMEM((CW, K), jnp.int32),
        pltpu.VMEM((CW, K), jnp.int32),
        pltpu.VMEM((K, D), jnp.float32),
        pltpu.VMEM((ZB, D), jnp.float32),
        pltpu.SemaphoreType.DMA,
        pltpu.VMEM_SHARED((A_ROWS, D), jnp.float32),
    ],
)


def _linear_block(a_ref, b_ref, w_ref, bias_ref, o_ref, *, relu):
    x = a_ref[...] + b_ref[...]
    y = lax.dot_general(x, w_ref[...], (((1,), (1,)), ((), ())),
                        preferred_element_type=jnp.float32)
    y = y + bias_ref[...]
    if relu:
        y = jnp.maximum(y, 0.0)
    o_ref[...] = y


def _linear(parts, w, bias, relu):
    blk = 2000
    return pl.pallas_call(
        functools.partial(_linear_block, relu=relu),
        grid=(N // blk,),
        in_specs=[
            pl.BlockSpec((blk, D), lambda i: (i, 0)),
            pl.BlockSpec((blk, D), lambda i: (i, 0)),
            pl.BlockSpec((D, D), lambda i: (0, 0)),
            pl.BlockSpec((1, D), lambda i: (0, 0)),
        ],
        out_specs=pl.BlockSpec((blk, D), lambda i: (i, 0)),
        out_shape=jax.ShapeDtypeStruct((N, D), jnp.float32),
    )(parts[0], parts[1], w, bias.reshape(1, D))


def kernel(h, edge_index, W1, b1, W2, b2):
    src = edge_index[0].astype(jnp.int32)
    dst = edge_index[1].astype(jnp.int32)
    pad = E_PAD - E
    srcs = jnp.concatenate([src, jnp.zeros((pad,), jnp.int32)]).reshape(NW, CW, K)
    dsts = jnp.concatenate([dst, jnp.full((pad,), DUMMY_DST, jnp.int32)]).reshape(NW, CW, K)

    parts0 = _segsum(h, srcs, dsts)
    h1 = _linear(parts0, W1, b1, relu=True)
    parts1 = _segsum(h1, srcs, dsts)
    return _linear(parts1, W2, b2, relu=False)


# speedup vs baseline: 13.1149x; 4.1855x over previous
import functools

import jax
import jax.numpy as jnp
from jax import lax
from jax.experimental import pallas as pl
from jax.experimental.pallas import tpu as pltpu
from jax.experimental.pallas import tpu_sc as plsc

N = 10000
E = 320000
D = 128

NC = 2
NS = 16
NW = NC * NS
K = 128
CW = 80
EPW = K * CW
E_PAD = EPW * NW

A_ROWS = 10240
ZROWS_PER_TILE = A_ROWS // NS
ZB = 32
HC = CW // 2
OUT_PER_TILE = 624
DUMMY_DST = N


def _segsum_body(table, srcs, dsts, out, src_v, dst_v, rows_v, zbuf,
                 sem0, sem1, accum):
    c = lax.axis_index("c")
    s = lax.axis_index("s")
    wid = s * NC + c

    zeros16 = jnp.zeros((16,), jnp.float32)

    def _zrow(i, _):
        def _zcol(j, __):
            zbuf[i, pl.ds(j * 16, 16)] = zeros16
            return 0
        lax.fori_loop(0, D // 16, _zcol, 0)
        return 0

    lax.fori_loop(0, ZB, _zrow, 0)

    def _zcopy(t, _):
        pltpu.sync_copy(zbuf, accum.at[pl.ds(s * ZROWS_PER_TILE + t * ZB, ZB)])
        return 0

    lax.fori_loop(0, ZROWS_PER_TILE // ZB, _zcopy, 0)

    plsc.subcore_barrier()

    sems = (sem0, sem1)

    def _half(h0):
        pltpu.sync_copy(srcs.at[wid, pl.ds(h0, HC)], src_v)
        pltpu.sync_copy(dsts.at[wid, pl.ds(h0, HC)], dst_v)
        pltpu.async_copy(table.at[src_v.at[0]], rows_v.at[0], sems[0])

        def _pair(i, _):
            for b in range(2):
                j = i * 2 + b

                @pl.when(j + 1 < HC)
                def _():
                    pltpu.async_copy(table.at[src_v.at[j + 1]],
                                     rows_v.at[1 - b], sems[1 - b])

                pltpu.make_async_copy(table.at[src_v.at[0]],
                                      rows_v.at[b], sems[b]).wait()
                pltpu.sync_copy(rows_v.at[b], accum.at[dst_v.at[j]], add=True)
            return 0

        lax.fori_loop(0, HC // 2, _pair, 0)

    _half(0)
    _half(HC)

    plsc.subcore_barrier()

    @pl.when(s < NS - 1)
    def _():
        pltpu.sync_copy(
            accum.at[pl.ds(s * OUT_PER_TILE, OUT_PER_TILE)],
            out.at[c, pl.ds(s * OUT_PER_TILE, OUT_PER_TILE)],
        )

    @pl.when(s == NS - 1)
    def _():
        tail = N - (NS - 1) * OUT_PER_TILE
        base = (NS - 1) * OUT_PER_TILE
        pltpu.sync_copy(
            accum.at[pl.ds(base, tail)],
            out.at[c, pl.ds(base, tail)],
        )


_segsum = pl.kernel(
    _segsum_body,
    out_type=jax.ShapeDtypeStruct((NC, N, D), jnp.float32),
    mesh=plsc.VectorSubcoreMesh(core_axis_name="c", subcore_axis_name="s"),
    scratch_types=[
        pltpu.VMEM((HC, K), jnp.int32),
        pltpu.VMEM((HC, K), jnp.int32),
        pltpu.VMEM((2, K, D), jnp.float32),
        pltpu.VMEM((ZB, D), jnp.float32),
        pltpu.SemaphoreType.DMA,
        pltpu.SemaphoreType.DMA,
        pltpu.VMEM_SHARED((A_ROWS, D), jnp.float32),
    ],
)


def _linear_block(a_ref, b_ref, w_ref, bias_ref, o_ref, *, relu):
    x = a_ref[...] + b_ref[...]
    y = lax.dot_general(x, w_ref[...], (((1,), (1,)), ((), ())),
                        preferred_element_type=jnp.float32)
    y = y + bias_ref[...]
    if relu:
        y = jnp.maximum(y, 0.0)
    o_ref[...] = y


def _linear(parts, w, bias, relu):
    blk = 2000
    nblk = N // blk
    flat = parts.reshape(NC * N, D)
    return pl.pallas_call(
        functools.partial(_linear_block, relu=relu),
        grid=(nblk,),
        in_specs=[
            pl.BlockSpec((blk, D), lambda i: (i, 0)),
            pl.BlockSpec((blk, D), lambda i: (i + nblk, 0)),
            pl.BlockSpec((D, D), lambda i: (0, 0)),
            pl.BlockSpec((1, D), lambda i: (0, 0)),
        ],
        out_specs=pl.BlockSpec((blk, D), lambda i: (i, 0)),
        out_shape=jax.ShapeDtypeStruct((N, D), jnp.float32),
    )(flat, flat, w, bias.reshape(1, D))


def kernel(h, edge_index, W1, b1, W2, b2):
    src = edge_index[0].astype(jnp.int32)
    dst = edge_index[1].astype(jnp.int32)
    pad = E_PAD - E
    pad_src = jnp.arange(pad, dtype=jnp.int32) % N
    pad_dst = DUMMY_DST + jnp.arange(pad, dtype=jnp.int32) % (A_ROWS - N)
    srcs = jnp.concatenate([src, pad_src]).reshape(NW, CW, K)
    dsts = jnp.concatenate([dst, pad_dst]).reshape(NW, CW, K)

    parts0 = _segsum(h, srcs, dsts)
    h1 = _linear(parts0, W1, b1, relu=True)
    parts1 = _segsum(h1, srcs, dsts)
    return _linear(parts1, W2, b2, relu=False)
